# baseline (speedup 1.0000x reference)
import functools

import jax
import jax.numpy as jnp
from jax import lax
from jax.experimental import pallas as pl
from jax.experimental.pallas import tpu as pltpu
from jax.experimental.pallas import tpu_sc as plsc

N = 10000
E = 320000
D = 128
H = 64
L = 16
NS = 16
K = 40
EPT = E // NS
NCH = EPT // K
NPT = N // NS
RN = 25
NNCH = NPT // RN
BN_INV = float(1.0 / (1.0 + 1e-5) ** 0.5)


def _proj_nodes(x, WA, bA, WB, bB, WD, bD, WE, bE):
    BR = 1000

    def body(x_ref, wa, ba, wb, bb, wd, bd, we, be, ax, bx, dx, ex):
        xb = x_ref[...]
        ax[...] = jnp.dot(xb, wa[...], preferred_element_type=jnp.float32) + ba[...]
        bx[...] = jnp.dot(xb, wb[...], preferred_element_type=jnp.float32) + bb[...]
        dx[...] = jnp.dot(xb, wd[...], preferred_element_type=jnp.float32) + bd[...]
        ex[...] = jnp.dot(xb, we[...], preferred_element_type=jnp.float32) + be[...]

    wspec = pl.BlockSpec((D, D), lambda i: (0, 0))
    bspec = pl.BlockSpec((1, D), lambda i: (0, 0))
    xspec = pl.BlockSpec((BR, D), lambda i: (i, 0))
    return pl.pallas_call(
        body,
        grid=(N // BR,),
        in_specs=[xspec, wspec, bspec, wspec, bspec, wspec, bspec, wspec, bspec],
        out_specs=[xspec, xspec, xspec, xspec],
        out_shape=[jax.ShapeDtypeStruct((N, D), jnp.float32)] * 4,
    )(x, WA, bA[None], WB, bB[None], WD, bD[None], WE, bE[None])


def _proj_edges(e, WC, bC):
    BR = 2000

    def body(e_ref, wc, bc, ce):
        ce[...] = jnp.dot(e_ref[...], wc[...], preferred_element_type=jnp.float32) + bc[...]

    return pl.pallas_call(
        body,
        grid=(E // BR,),
        in_specs=[
            pl.BlockSpec((BR, D), lambda i: (i, 0)),
            pl.BlockSpec((D, D), lambda i: (0, 0)),
            pl.BlockSpec((1, D), lambda i: (0, 0)),
        ],
        out_specs=pl.BlockSpec((BR, D), lambda i: (i, 0)),
        out_shape=jax.ShapeDtypeStruct((E, D), jnp.float32),
    )(e, WC, bC[None])


def _sc_gnn(ax, bxt, dxt, ext, ce, e, x, dst, d2, s2, gxs, bxb, ges, beb):
    mesh = plsc.VectorSubcoreMesh(core_axis_name="c", subcore_axis_name="s")

    @functools.partial(
        pl.kernel,
        out_type=[
            jax.ShapeDtypeStruct((N, D), jnp.float32),
            jax.ShapeDtypeStruct((E, D), jnp.float32),
        ],
        mesh=mesh,
        compiler_params=pltpu.CompilerParams(use_tc_tiling_on_sc=False,
                                             needs_layout_passes=False),
        scratch_types=[
            pltpu.VMEM((2, K), jnp.int32),
            pltpu.VMEM((2, 2, K), jnp.int32),
            pltpu.VMEM((2, K, H), jnp.bfloat16),
            pltpu.VMEM((2, K, H), jnp.bfloat16),
            pltpu.VMEM((2, K, H), jnp.bfloat16),
            pltpu.VMEM((2, K, H), jnp.float32),
            pltpu.VMEM((2, K, H), jnp.float32),
            pltpu.VMEM((2, K, H), jnp.float32),
            pltpu.VMEM((2, K, H), jnp.float32),
            pltpu.VMEM((2, K, H), jnp.float32),
            pltpu.VMEM((RN, H), jnp.float32),
            pltpu.VMEM((RN, H), jnp.float32),
            pltpu.VMEM((RN, H), jnp.float32),
            pltpu.VMEM((RN, H), jnp.float32),
            pltpu.VMEM((RN, H), jnp.float32),
            pltpu.VMEM((H,), jnp.float32),
            pltpu.VMEM((H,), jnp.float32),
            pltpu.VMEM((H,), jnp.float32),
            pltpu.VMEM((H,), jnp.float32),
            pltpu.VMEM_SHARED((N, H), jnp.float32),
            pltpu.VMEM_SHARED((N, H), jnp.float32),
            pltpu.SemaphoreType.DMA,
            pltpu.SemaphoreType.DMA,
            pltpu.SemaphoreType.DMA,
            pltpu.SemaphoreType.DMA,
            pltpu.SemaphoreType.DMA,
            pltpu.SemaphoreType.DMA,
            pltpu.SemaphoreType.DMA,
            pltpu.SemaphoreType.DMA,
            pltpu.SemaphoreType.DMA,
            pltpu.SemaphoreType.DMA,
        ],
    )
    def k(ax_h, bxt_h, dxt_h, ext_h, ce_h, e_h, x_h, dst_h, d2_h, s2_h,
          gxs_h, bxb_h, ges_h, beb_h, xout_h, eout_h,
          dstb, gidx, gD, gE, gB, vnum, ceb, ebuf, obuf, sgb,
          anb, adb, axb, xbuf, xob, gxs_v, bxv, gesv, bev,
          accN, accD,
          sem_gi0, sem_gi1, sem_ld0, sem_ld1, sem_st0, sem_st1,
          sem_dst0, sem_dst1, sem_lin0, sem_lin1):
        c = lax.axis_index("c")
        s = lax.axis_index("s")
        n0t = s * NPT
        col0 = c * H
        cE = c * E
        ebase = s * EPT
        sem_gi = (sem_gi0, sem_gi1)
        sem_ld = (sem_ld0, sem_ld1)
        sem_st = (sem_st0, sem_st1)
        sem_dst = (sem_dst0, sem_dst1)
        sem_lin = (sem_lin0, sem_lin1)

        def zrow(kk, carry):
            for l in range(H // L):
                anb[kk, pl.ds(l * L, L)] = jnp.zeros((L,), jnp.float32)
                adb[kk, pl.ds(l * L, L)] = jnp.zeros((L,), jnp.float32)
            return carry

        lax.fori_loop(0, RN, zrow, 0)
        for j in range(NNCH):
            pltpu.sync_copy(anb, accN.at[pl.ds(n0t + j * RN, RN)])
            pltpu.sync_copy(adb, accD.at[pl.ds(n0t + j * RN, RN)])

        pltpu.sync_copy(gxs_h.at[pl.ds(col0, H)], gxs_v)
        pltpu.sync_copy(bxb_h.at[pl.ds(col0, H)], bxv)
        pltpu.sync_copy(ges_h.at[pl.ds(col0, H)], gesv)
        pltpu.sync_copy(beb_h.at[pl.ds(col0, H)], bev)
        plsc.subcore_barrier()

        def issue_gidx(chi, bb):
            base = cE + ebase + chi * K
            pltpu.async_copy(d2_h.at[pl.ds(base, K)], gidx.at[bb].at[0],
                             sem_gi[bb])
            pltpu.async_copy(s2_h.at[pl.ds(base, K)], gidx.at[bb].at[1],
                             sem_gi[bb])

        def wait_gidx(bb):
            pltpu.make_async_copy(d2_h.at[pl.ds(0, K)], gidx.at[bb].at[0],
                                  sem_gi[bb]).wait()
            pltpu.make_async_copy(s2_h.at[pl.ds(0, K)], gidx.at[bb].at[1],
                                  sem_gi[bb]).wait()

        def issue_loads(chi, bb):
            base = ebase + chi * K
            pltpu.async_copy(dxt_h.at[gidx.at[bb].at[0]], gD.at[bb], sem_ld[bb])
            pltpu.async_copy(ext_h.at[gidx.at[bb].at[1]], gE.at[bb], sem_ld[bb])
            pltpu.async_copy(bxt_h.at[gidx.at[bb].at[1]], gB.at[bb], sem_ld[bb])
            pltpu.async_copy(ce_h.at[pl.ds(base, K), pl.ds(col0, H)],
                             ceb.at[bb], sem_lin[bb])
            pltpu.async_copy(e_h.at[pl.ds(base, K), pl.ds(col0, H)],
                             ebuf.at[bb], sem_lin[bb])

        def wait_loads(bb):
            pltpu.make_async_copy(dxt_h.at[gidx.at[bb].at[0]], gD.at[bb],
                                  sem_ld[bb]).wait()
            pltpu.make_async_copy(ext_h.at[gidx.at[bb].at[1]], gE.at[bb],
                                  sem_ld[bb]).wait()
            pltpu.make_async_copy(bxt_h.at[gidx.at[bb].at[1]], gB.at[bb],
                                  sem_ld[bb]).wait()
            pltpu.make_async_copy(ce_h.at[pl.ds(0, K), pl.ds(0, H)],
                                  ceb.at[bb], sem_lin[bb]).wait()
            pltpu.make_async_copy(e_h.at[pl.ds(0, K), pl.ds(0, H)],
                                  ebuf.at[bb], sem_lin[bb]).wait()

        def wait_eout(bb):
            pltpu.make_async_copy(eout_h.at[pl.ds(0, K), pl.ds(0, H)],
                                  obuf.at[bb], sem_st[bb]).wait()

        gev = tuple(gesv[pl.ds(l * L, L)] for l in range(H // L))
        bevr = tuple(bev[pl.ds(l * L, L)] for l in range(H // L))

        issue_gidx(0, 0)
        issue_gidx(1, 1)
        wait_gidx(0)
        issue_loads(0, 0)
        pltpu.async_copy(dst_h.at[pl.ds(ebase, K)], dstb.at[0], sem_dst[0])

        def step(i2, carry):
            for b in (0, 1):
                i = i2 * 2 + b
                base = ebase + i * K

                @pl.when(i <= NCH - 2)
                def _():
                    wait_gidx(1 - b)
                    issue_loads(i + 1, 1 - b)
                    pltpu.async_copy(dst_h.at[pl.ds(base + K, K)],
                                     dstb.at[1 - b], sem_dst[1 - b])

                wait_loads(b)

                @pl.when(i <= NCH - 3)
                def _():
                    issue_gidx(i + 2, b)

                @pl.when(i2 >= 1)
                def _():
                    wait_eout(b)

                def row(kk, rcarry):
                    for g in range(H // 32):
                        sg32 = pl.ds(g * 32, 32)
                        d0, d1 = plsc.unpack(gD[b, kk, sg32],
                                             format=plsc.PackFormat.INTERLEAVED)
                        e0, e1 = plsc.unpack(gE[b, kk, sg32],
                                             format=plsc.PackFormat.INTERLEAVED)
                        b0, b1 = plsc.unpack(gB[b, kk, sg32],
                                             format=plsc.PackFormat.INTERLEAVED)
                        for h, dv, ev, bv in ((0, d0, e0, b0), (1, d1, e1, b1)):
                            l = g * 2 + h
                            sl = pl.ds(l * L, L)
                            t = dv + ev + ceb[b, kk, sl]
                            sg = 1.0 / (1.0 + jnp.exp(-t))
                            vnum[b, kk, sl] = sg * bv
                            sgb[b, kk, sl] = sg
                            obuf[b, kk, sl] = ebuf[b, kk, sl] + jnp.maximum(
                                t * gev[l] + bevr[l], 0.0)
                    return rcarry

                lax.fori_loop(0, K, row, 0)

                pltpu.make_async_copy(dst_h.at[pl.ds(base, K)], dstb.at[b],
                                      sem_dst[b]).wait()
                pltpu.sync_copy(vnum.at[b], accN.at[dstb.at[b]], add=True)
                pltpu.sync_copy(sgb.at[b], accD.at[dstb.at[b]], add=True)
                pltpu.async_copy(obuf.at[b],
                                 eout_h.at[pl.ds(base, K), pl.ds(col0, H)],
                                 sem_st[b])
            return carry

        lax.fori_loop(0, NCH // 2, step, 0)
        wait_eout(0)
        wait_eout(1)
        plsc.subcore_barrier()

        gxv = tuple(gxs_v[pl.ds(l * L, L)] for l in range(H // L))
        bxvr = tuple(bxv[pl.ds(l * L, L)] for l in range(H // L))
        for j in range(NNCH):
            n0 = n0t + j * RN
            pltpu.sync_copy(accN.at[pl.ds(n0, RN)], anb)
            pltpu.sync_copy(accD.at[pl.ds(n0, RN)], adb)
            pltpu.sync_copy(ax_h.at[pl.ds(n0, RN), pl.ds(col0, H)], axb)
            pltpu.sync_copy(x_h.at[pl.ds(n0, RN), pl.ds(col0, H)], xbuf)

            def nrow(kk, rcarry):
                for l in range(H // L):
                    sl = pl.ds(l * L, L)
                    aggr = anb[kk, sl] / (adb[kk, sl] + 1e-6)
                    u = axb[kk, sl] + aggr
                    xob[kk, sl] = xbuf[kk, sl] + jnp.maximum(
                        u * gxv[l] + bxvr[l], 0.0)
                return rcarry

            lax.fori_loop(0, RN, nrow, 0)
            pltpu.sync_copy(xob, xout_h.at[pl.ds(n0, RN), pl.ds(col0, H)])

    return k(ax, bxt, dxt, ext, ce, e, x, dst, d2, s2, gxs, bxb, ges, beb)


def kernel(x, e, edge_index, WA, bA, WB, bB, WC, bC, WD, bD, WE, bE,
           gamma_x, beta_x, gamma_e, beta_e):
    src = edge_index[0].astype(jnp.int32)
    dst = edge_index[1].astype(jnp.int32)
    d2 = jnp.concatenate([2 * dst, 2 * dst + 1])
    s2 = jnp.concatenate([2 * src, 2 * src + 1])

    ax, bx, dx, ex = _proj_nodes(x, WA, bA, WB, bB, WD, bD, WE, bE)
    ce = _proj_edges(e, WC, bC)

    def _pack_perm(t):
        tb = t.astype(jnp.bfloat16)
        return tb.reshape(N, 2, 2, 2, L).transpose(0, 1, 2, 4, 3).reshape(
            2 * N, H)

    bxt = _pack_perm(bx)
    dxt = _pack_perm(dx)
    ext = _pack_perm(ex)
    gxs = gamma_x * jnp.float32(BN_INV)
    ges = gamma_e * jnp.float32(BN_INV)

    x_out, e_out = _sc_gnn(ax, bxt, dxt, ext, ce, e, x, dst, d2, s2,
                           gxs, beta_x, ges, beta_e)
    return x_out, e_out


# speedup vs baseline: 1.0934x; 1.0934x over previous
import functools

import jax
import jax.numpy as jnp
from jax import lax
from jax.experimental import pallas as pl
from jax.experimental.pallas import tpu as pltpu
from jax.experimental.pallas import tpu_sc as plsc

N = 10000
E = 320000
D = 128
H = 64
L = 16
NS = 16
K = 40
EPT = E // NS
NCH = EPT // K
NPT = N // NS
RN = 25
NNCH = NPT // RN
BN_INV = float(1.0 / (1.0 + 1e-5) ** 0.5)


def _proj_nodes(x, WA, bA, WB, bB, WD, bD, WE, bE):
    BR = 1000

    def body(x_ref, wa, ba, wb, bb, wd, bd, we, be, ax, bx, dx, ex):
        xb = x_ref[...]
        ax[...] = jnp.dot(xb, wa[...], preferred_element_type=jnp.float32) + ba[...]
        bx[...] = jnp.dot(xb, wb[...], preferred_element_type=jnp.float32) + bb[...]
        dx[...] = jnp.dot(xb, wd[...], preferred_element_type=jnp.float32) + bd[...]
        ex[...] = jnp.dot(xb, we[...], preferred_element_type=jnp.float32) + be[...]

    wspec = pl.BlockSpec((D, D), lambda i: (0, 0))
    bspec = pl.BlockSpec((1, D), lambda i: (0, 0))
    xspec = pl.BlockSpec((BR, D), lambda i: (i, 0))
    return pl.pallas_call(
        body,
        grid=(N // BR,),
        in_specs=[xspec, wspec, bspec, wspec, bspec, wspec, bspec, wspec, bspec],
        out_specs=[xspec, xspec, xspec, xspec],
        out_shape=[jax.ShapeDtypeStruct((N, D), jnp.float32)] * 4,
    )(x, WA, bA[None], WB, bB[None], WD, bD[None], WE, bE[None])


def _proj_edges(e, WC, bC):
    BR = 2000

    def body(e_ref, wc, bc, ce):
        ce[...] = jnp.dot(e_ref[...], wc[...], preferred_element_type=jnp.float32) + bc[...]

    return pl.pallas_call(
        body,
        grid=(E // BR,),
        in_specs=[
            pl.BlockSpec((BR, D), lambda i: (i, 0)),
            pl.BlockSpec((D, D), lambda i: (0, 0)),
            pl.BlockSpec((1, D), lambda i: (0, 0)),
        ],
        out_specs=pl.BlockSpec((BR, D), lambda i: (i, 0)),
        out_shape=jax.ShapeDtypeStruct((E, D), jnp.float32),
    )(e, WC, bC[None])


def _sc_gnn(ax, ebxt, dxt, ce, e, x, dst, d2, s2, gxs, bxb, ges, beb):
    mesh = plsc.VectorSubcoreMesh(core_axis_name="c", subcore_axis_name="s")

    @functools.partial(
        pl.kernel,
        out_type=[
            jax.ShapeDtypeStruct((N, D), jnp.float32),
            jax.ShapeDtypeStruct((E, D), jnp.float32),
        ],
        mesh=mesh,
        compiler_params=pltpu.CompilerParams(use_tc_tiling_on_sc=False),
        scratch_types=[
            pltpu.VMEM((2, K), jnp.int32),
            pltpu.VMEM((2, 2, K), jnp.int32),
            pltpu.VMEM((2, K, H), jnp.float32),
            pltpu.VMEM((2, K, D), jnp.float32),
            pltpu.VMEM((2, K, H), jnp.float32),
            pltpu.VMEM((2, K, H), jnp.float32),
            pltpu.VMEM((2, K, H), jnp.float32),
            pltpu.VMEM((2, K, H), jnp.float32),
            pltpu.VMEM((2, K, H), jnp.float32),
            pltpu.VMEM((RN, H), jnp.float32),
            pltpu.VMEM((RN, H), jnp.float32),
            pltpu.VMEM((RN, H), jnp.float32),
            pltpu.VMEM((RN, H), jnp.float32),
            pltpu.VMEM((RN, H), jnp.float32),
            pltpu.VMEM((H,), jnp.float32),
            pltpu.VMEM((H,), jnp.float32),
            pltpu.VMEM((H,), jnp.float32),
            pltpu.VMEM((H,), jnp.float32),
            pltpu.VMEM_SHARED((N, H), jnp.float32),
            pltpu.VMEM_SHARED((N, H), jnp.float32),
            pltpu.SemaphoreType.DMA,
            pltpu.SemaphoreType.DMA,
            pltpu.SemaphoreType.DMA,
            pltpu.SemaphoreType.DMA,
            pltpu.SemaphoreType.DMA,
            pltpu.SemaphoreType.DMA,
            pltpu.SemaphoreType.DMA,
            pltpu.SemaphoreType.DMA,
            pltpu.SemaphoreType.DMA,
            pltpu.SemaphoreType.DMA,
        ],
    )
    def k(ax_h, ebxt_h, dxt_h, ce_h, e_h, x_h, dst_h, d2_h, s2_h,
          gxs_h, bxb_h, ges_h, beb_h, xout_h, eout_h,
          dstb, gidx, gD, geb, vnum, ceb, ebuf, obuf, sgb,
          anb, adb, axb, xbuf, xob, gxs_v, bxv, gesv, bev,
          accN, accD,
          sem_gi0, sem_gi1, sem_ld0, sem_ld1, sem_st0, sem_st1,
          sem_dst0, sem_dst1, sem_lin0, sem_lin1):
        c = lax.axis_index("c")
        s = lax.axis_index("s")
        n0t = s * NPT
        col0 = c * H
        cE = c * E
        ebase = s * EPT
        sem_gi = (sem_gi0, sem_gi1)
        sem_ld = (sem_ld0, sem_ld1)
        sem_st = (sem_st0, sem_st1)
        sem_dst = (sem_dst0, sem_dst1)
        sem_lin = (sem_lin0, sem_lin1)

        def zrow(kk, carry):
            for l in range(H // L):
                anb[kk, pl.ds(l * L, L)] = jnp.zeros((L,), jnp.float32)
                adb[kk, pl.ds(l * L, L)] = jnp.zeros((L,), jnp.float32)
            return carry

        lax.fori_loop(0, RN, zrow, 0)
        for j in range(NNCH):
            pltpu.sync_copy(anb, accN.at[pl.ds(n0t + j * RN, RN)])
            pltpu.sync_copy(adb, accD.at[pl.ds(n0t + j * RN, RN)])

        pltpu.sync_copy(gxs_h.at[pl.ds(col0, H)], gxs_v)
        pltpu.sync_copy(bxb_h.at[pl.ds(col0, H)], bxv)
        pltpu.sync_copy(ges_h.at[pl.ds(col0, H)], gesv)
        pltpu.sync_copy(beb_h.at[pl.ds(col0, H)], bev)
        plsc.subcore_barrier()

        def issue_gidx(chi, bb):
            base = cE + ebase + chi * K
            pltpu.async_copy(d2_h.at[pl.ds(base, K)], gidx.at[bb].at[0],
                             sem_gi[bb])
            pltpu.async_copy(s2_h.at[pl.ds(base, K)], gidx.at[bb].at[1],
                             sem_gi[bb])

        def wait_gidx(bb):
            pltpu.make_async_copy(d2_h.at[pl.ds(0, K)], gidx.at[bb].at[0],
                                  sem_gi[bb]).wait()
            pltpu.make_async_copy(s2_h.at[pl.ds(0, K)], gidx.at[bb].at[1],
                                  sem_gi[bb]).wait()

        def issue_loads(chi, bb):
            base = ebase + chi * K
            pltpu.async_copy(dxt_h.at[gidx.at[bb].at[0]], gD.at[bb], sem_ld[bb])
            pltpu.async_copy(ebxt_h.at[gidx.at[bb].at[1]], geb.at[bb],
                             sem_ld[bb])
            pltpu.async_copy(ce_h.at[pl.ds(base, K), pl.ds(col0, H)],
                             ceb.at[bb], sem_lin[bb])
            pltpu.async_copy(e_h.at[pl.ds(base, K), pl.ds(col0, H)],
                             ebuf.at[bb], sem_lin[bb])

        def wait_loads(bb):
            pltpu.make_async_copy(dxt_h.at[gidx.at[bb].at[0]], gD.at[bb],
                                  sem_ld[bb]).wait()
            pltpu.make_async_copy(ebxt_h.at[gidx.at[bb].at[1]], geb.at[bb],
                                  sem_ld[bb]).wait()
            pltpu.make_async_copy(ce_h.at[pl.ds(0, K), pl.ds(0, H)],
                                  ceb.at[bb], sem_lin[bb]).wait()
            pltpu.make_async_copy(e_h.at[pl.ds(0, K), pl.ds(0, H)],
                                  ebuf.at[bb], sem_lin[bb]).wait()

        def wait_eout(bb):
            pltpu.make_async_copy(eout_h.at[pl.ds(0, K), pl.ds(0, H)],
                                  obuf.at[bb], sem_st[bb]).wait()

        gev = tuple(gesv[pl.ds(l * L, L)] for l in range(H // L))
        bevr = tuple(bev[pl.ds(l * L, L)] for l in range(H // L))

        issue_gidx(0, 0)
        issue_gidx(1, 1)
        wait_gidx(0)
        issue_loads(0, 0)
        pltpu.async_copy(dst_h.at[pl.ds(ebase, K)], dstb.at[0], sem_dst[0])

        def step(i2, carry):
            for b in (0, 1):
                i = i2 * 2 + b
                base = ebase + i * K

                @pl.when(i <= NCH - 2)
                def _():
                    wait_gidx(1 - b)
                    issue_loads(i + 1, 1 - b)
                    pltpu.async_copy(dst_h.at[pl.ds(base + K, K)],
                                     dstb.at[1 - b], sem_dst[1 - b])

                wait_loads(b)

                @pl.when(i <= NCH - 3)
                def _():
                    issue_gidx(i + 2, b)

                @pl.when(i2 >= 1)
                def _():
                    wait_eout(b)

                def row(kk, rcarry):
                    for l in range(H // L):
                        sl = pl.ds(l * L, L)
                        t = gD[b, kk, sl] + geb[b, kk, sl] + ceb[b, kk, sl]
                        sg = 1.0 / (1.0 + jnp.exp(-t))
                        vnum[b, kk, sl] = sg * geb[b, kk, pl.ds(H + l * L, L)]
                        sgb[b, kk, sl] = sg
                        obuf[b, kk, sl] = ebuf[b, kk, sl] + jnp.maximum(
                            t * gev[l] + bevr[l], 0.0)
                    return rcarry

                lax.fori_loop(0, K, row, 0)

                pltpu.make_async_copy(dst_h.at[pl.ds(base, K)], dstb.at[b],
                                      sem_dst[b]).wait()
                pltpu.sync_copy(vnum.at[b], accN.at[dstb.at[b]], add=True)
                pltpu.sync_copy(sgb.at[b], accD.at[dstb.at[b]], add=True)
                pltpu.async_copy(obuf.at[b],
                                 eout_h.at[pl.ds(base, K), pl.ds(col0, H)],
                                 sem_st[b])
            return carry

        lax.fori_loop(0, NCH // 2, step, 0)
        wait_eout(0)
        wait_eout(1)
        plsc.subcore_barrier()

        gxv = tuple(gxs_v[pl.ds(l * L, L)] for l in range(H // L))
        bxvr = tuple(bxv[pl.ds(l * L, L)] for l in range(H // L))
        for j in range(NNCH):
            n0 = n0t + j * RN
            pltpu.sync_copy(accN.at[pl.ds(n0, RN)], anb)
            pltpu.sync_copy(accD.at[pl.ds(n0, RN)], adb)
            pltpu.sync_copy(ax_h.at[pl.ds(n0, RN), pl.ds(col0, H)], axb)
            pltpu.sync_copy(x_h.at[pl.ds(n0, RN), pl.ds(col0, H)], xbuf)

            def nrow(kk, rcarry):
                for l in range(H // L):
                    sl = pl.ds(l * L, L)
                    aggr = anb[kk, sl] / (adb[kk, sl] + 1e-6)
                    u = axb[kk, sl] + aggr
                    xob[kk, sl] = xbuf[kk, sl] + jnp.maximum(
                        u * gxv[l] + bxvr[l], 0.0)
                return rcarry

            lax.fori_loop(0, RN, nrow, 0)
            pltpu.sync_copy(xob, xout_h.at[pl.ds(n0, RN), pl.ds(col0, H)])

    return k(ax, ebxt, dxt, ce, e, x, dst, d2, s2, gxs, bxb, ges, beb)


def kernel(x, e, edge_index, WA, bA, WB, bB, WC, bC, WD, bD, WE, bE,
           gamma_x, beta_x, gamma_e, beta_e):
    src = edge_index[0].astype(jnp.int32)
    dst = edge_index[1].astype(jnp.int32)
    d2 = jnp.concatenate([2 * dst, 2 * dst + 1])
    s2 = jnp.concatenate([2 * src, 2 * src + 1])

    ax, bx, dx, ex = _proj_nodes(x, WA, bA, WB, bB, WD, bD, WE, bE)
    ce = _proj_edges(e, WC, bC)

    ebxt = jnp.concatenate(
        [ex.reshape(N, 2, H), bx.reshape(N, 2, H)], axis=2).reshape(2 * N, D)
    dxt = dx.reshape(2 * N, H)
    gxs = gamma_x * jnp.float32(BN_INV)
    ges = gamma_e * jnp.float32(BN_INV)

    x_out, e_out = _sc_gnn(ax, ebxt, dxt, ce, e, x, dst, d2, s2,
                           gxs, beta_x, ges, beta_e)
    return x_out, e_out


# speedup vs baseline: 3.0840x; 2.8206x over previous
import functools

import jax
import jax.numpy as jnp
from jax import lax
from jax.experimental import pallas as pl
from jax.experimental.pallas import tpu as pltpu
from jax.experimental.pallas import tpu_sc as plsc

N = 10000
E = 320000
D = 128
H = 64
L = 16
NS = 16
K = 40
EPT = E // NS
NCH = EPT // K
NPT = N // NS
RN = 25
NNCH = NPT // RN
BN_INV = float(1.0 / (1.0 + 1e-5) ** 0.5)


def _proj_nodes(x, WA, bA, WB, bB, WD, bD, WE, bE):
    BR = 1000

    def body(x_ref, wa, ba, wb, bb, wd, bd, we, be, ax, bx, dx, ex):
        xb = x_ref[...]
        ax[...] = jnp.dot(xb, wa[...], preferred_element_type=jnp.float32) + ba[...]
        bx[...] = jnp.dot(xb, wb[...], preferred_element_type=jnp.float32) + bb[...]
        dx[...] = jnp.dot(xb, wd[...], preferred_element_type=jnp.float32) + bd[...]
        ex[...] = jnp.dot(xb, we[...], preferred_element_type=jnp.float32) + be[...]

    wspec = pl.BlockSpec((D, D), lambda i: (0, 0))
    bspec = pl.BlockSpec((1, D), lambda i: (0, 0))
    xspec = pl.BlockSpec((BR, D), lambda i: (i, 0))
    return pl.pallas_call(
        body,
        grid=(N // BR,),
        in_specs=[xspec, wspec, bspec, wspec, bspec, wspec, bspec, wspec, bspec],
        out_specs=[xspec, xspec, xspec, xspec],
        out_shape=[jax.ShapeDtypeStruct((N, D), jnp.float32)] * 4,
    )(x, WA, bA[None], WB, bB[None], WD, bD[None], WE, bE[None])


def _proj_edges(e, WC, bC):
    BR = 2000

    def body(e_ref, wc, bc, ce):
        ce[...] = jnp.dot(e_ref[...], wc[...], preferred_element_type=jnp.float32) + bc[...]

    return pl.pallas_call(
        body,
        grid=(E // BR,),
        in_specs=[
            pl.BlockSpec((BR, D), lambda i: (i, 0)),
            pl.BlockSpec((D, D), lambda i: (0, 0)),
            pl.BlockSpec((1, D), lambda i: (0, 0)),
        ],
        out_specs=pl.BlockSpec((BR, D), lambda i: (i, 0)),
        out_shape=jax.ShapeDtypeStruct((E, D), jnp.float32),
    )(e, WC, bC[None])


def _sc_gnn(ax, bxt, dxt, ext, ce, e, x, dst, d2, s2, gxs, bxb, ges, beb):
    mesh = plsc.VectorSubcoreMesh(core_axis_name="c", subcore_axis_name="s")

    @functools.partial(
        pl.kernel,
        out_type=[
            jax.ShapeDtypeStruct((N, D), jnp.float32),
            jax.ShapeDtypeStruct((E, D), jnp.float32),
        ],
        mesh=mesh,
        compiler_params=pltpu.CompilerParams(use_tc_tiling_on_sc=False),
        scratch_types=[
            pltpu.VMEM((2, K), jnp.int32),
            pltpu.VMEM((2, 2, K), jnp.int32),
            pltpu.VMEM((2, K, H), jnp.float32),
            pltpu.VMEM((2, K, H), jnp.float32),
            pltpu.VMEM((2, K, H), jnp.float32),
            pltpu.VMEM((2, K, H), jnp.float32),
            pltpu.VMEM((2, K, H), jnp.float32),
            pltpu.VMEM((2, K, H), jnp.float32),
            pltpu.VMEM((2, K, H), jnp.float32),
            pltpu.VMEM((2, K, H), jnp.float32),
            pltpu.VMEM((RN, H), jnp.float32),
            pltpu.VMEM((RN, H), jnp.float32),
            pltpu.VMEM((RN, H), jnp.float32),
            pltpu.VMEM((RN, H), jnp.float32),
            pltpu.VMEM((RN, H), jnp.float32),
            pltpu.VMEM((H,), jnp.float32),
            pltpu.VMEM((H,), jnp.float32),
            pltpu.VMEM((H,), jnp.float32),
            pltpu.VMEM((H,), jnp.float32),
            pltpu.VMEM_SHARED((N, H), jnp.float32),
            pltpu.VMEM_SHARED((N, H), jnp.float32),
            pltpu.SemaphoreType.DMA,
            pltpu.SemaphoreType.DMA,
            pltpu.SemaphoreType.DMA,
            pltpu.SemaphoreType.DMA,
            pltpu.SemaphoreType.DMA,
            pltpu.SemaphoreType.DMA,
            pltpu.SemaphoreType.DMA,
            pltpu.SemaphoreType.DMA,
            pltpu.SemaphoreType.DMA,
            pltpu.SemaphoreType.DMA,
        ],
    )
    def k(ax_h, bxt_h, dxt_h, ext_h, ce_h, e_h, x_h, dst_h, d2_h, s2_h,
          gxs_h, bxb_h, ges_h, beb_h, xout_h, eout_h,
          dstb, gidx, gD, gE, gB, vnum, ceb, ebuf, obuf, sgb,
          anb, adb, axb, xbuf, xob, gxs_v, bxv, gesv, bev,
          accN, accD,
          sem_gi0, sem_gi1, sem_ld0, sem_ld1, sem_st0, sem_st1,
          sem_dst0, sem_dst1, sem_lin0, sem_lin1):
        c = lax.axis_index("c")
        s = lax.axis_index("s")
        n0t = s * NPT
        col0 = c * H
        cE = c * E
        ebase = s * EPT
        sem_gi = (sem_gi0, sem_gi1)
        sem_ld = (sem_ld0, sem_ld1)
        sem_st = (sem_st0, sem_st1)
        sem_dst = (sem_dst0, sem_dst1)
        sem_lin = (sem_lin0, sem_lin1)

        def zrow(kk, carry):
            for l in range(H // L):
                anb[kk, pl.ds(l * L, L)] = jnp.zeros((L,), jnp.float32)
                adb[kk, pl.ds(l * L, L)] = jnp.zeros((L,), jnp.float32)
            return carry

        lax.fori_loop(0, RN, zrow, 0)
        for j in range(NNCH):
            pltpu.sync_copy(anb, accN.at[pl.ds(n0t + j * RN, RN)])
            pltpu.sync_copy(adb, accD.at[pl.ds(n0t + j * RN, RN)])

        pltpu.sync_copy(gxs_h.at[pl.ds(col0, H)], gxs_v)
        pltpu.sync_copy(bxb_h.at[pl.ds(col0, H)], bxv)
        pltpu.sync_copy(ges_h.at[pl.ds(col0, H)], gesv)
        pltpu.sync_copy(beb_h.at[pl.ds(col0, H)], bev)
        plsc.subcore_barrier()

        def issue_gidx(chi, bb):
            base = cE + ebase + chi * K
            pltpu.async_copy(d2_h.at[pl.ds(base, K)], gidx.at[bb].at[0],
                             sem_gi[bb])
            pltpu.async_copy(s2_h.at[pl.ds(base, K)], gidx.at[bb].at[1],
                             sem_gi[bb])

        def wait_gidx(bb):
            pltpu.make_async_copy(d2_h.at[pl.ds(0, K)], gidx.at[bb].at[0],
                                  sem_gi[bb]).wait()
            pltpu.make_async_copy(s2_h.at[pl.ds(0, K)], gidx.at[bb].at[1],
                                  sem_gi[bb]).wait()

        def issue_loads(chi, bb):
            base = ebase + chi * K
            pltpu.async_copy(dxt_h.at[gidx.at[bb].at[0]], gD.at[bb], sem_ld[bb])
            pltpu.async_copy(ext_h.at[gidx.at[bb].at[1]], gE.at[bb], sem_ld[bb])
            pltpu.async_copy(bxt_h.at[gidx.at[bb].at[1]], gB.at[bb], sem_ld[bb])
            pltpu.async_copy(ce_h.at[pl.ds(base, K), pl.ds(col0, H)],
                             ceb.at[bb], sem_lin[bb])
            pltpu.async_copy(e_h.at[pl.ds(base, K), pl.ds(col0, H)],
                             ebuf.at[bb], sem_lin[bb])

        def wait_loads(bb):
            pltpu.make_async_copy(dxt_h.at[gidx.at[bb].at[0]], gD.at[bb],
                                  sem_ld[bb]).wait()
            pltpu.make_async_copy(ext_h.at[gidx.at[bb].at[1]], gE.at[bb],
                                  sem_ld[bb]).wait()
            pltpu.make_async_copy(bxt_h.at[gidx.at[bb].at[1]], gB.at[bb],
                                  sem_ld[bb]).wait()
            pltpu.make_async_copy(ce_h.at[pl.ds(0, K), pl.ds(0, H)],
                                  ceb.at[bb], sem_lin[bb]).wait()
            pltpu.make_async_copy(e_h.at[pl.ds(0, K), pl.ds(0, H)],
                                  ebuf.at[bb], sem_lin[bb]).wait()

        def wait_eout(bb):
            pltpu.make_async_copy(eout_h.at[pl.ds(0, K), pl.ds(0, H)],
                                  obuf.at[bb], sem_st[bb]).wait()

        gev = tuple(gesv[pl.ds(l * L, L)] for l in range(H // L))
        bevr = tuple(bev[pl.ds(l * L, L)] for l in range(H // L))

        issue_gidx(0, 0)
        issue_gidx(1, 1)
        wait_gidx(0)
        issue_loads(0, 0)
        pltpu.async_copy(dst_h.at[pl.ds(ebase, K)], dstb.at[0], sem_dst[0])

        def step(i2, carry):
            for b in (0, 1):
                i = i2 * 2 + b
                base = ebase + i * K

                @pl.when(i <= NCH - 2)
                def _():
                    wait_gidx(1 - b)
                    issue_loads(i + 1, 1 - b)
                    pltpu.async_copy(dst_h.at[pl.ds(base + K, K)],
                                     dstb.at[1 - b], sem_dst[1 - b])

                wait_loads(b)

                @pl.when(i <= NCH - 3)
                def _():
                    issue_gidx(i + 2, b)

                @pl.when(i2 >= 1)
                def _():
                    wait_eout(b)

                def row(kk, rcarry):
                    for l in range(H // L):
                        sl = pl.ds(l * L, L)
                        t = gD[b, kk, sl] + gE[b, kk, sl] + ceb[b, kk, sl]
                        sg = 1.0 / (1.0 + jnp.exp(-t))
                        vnum[b, kk, sl] = sg * gB[b, kk, sl]
                        sgb[b, kk, sl] = sg
                        obuf[b, kk, sl] = ebuf[b, kk, sl] + jnp.maximum(
                            t * gev[l] + bevr[l], 0.0)
                    return rcarry

                lax.fori_loop(0, K, row, 0)

                pltpu.make_async_copy(dst_h.at[pl.ds(base, K)], dstb.at[b],
                                      sem_dst[b]).wait()
                pltpu.sync_copy(vnum.at[b], accN.at[dstb.at[b]], add=True)
                pltpu.sync_copy(sgb.at[b], accD.at[dstb.at[b]], add=True)
                pltpu.async_copy(obuf.at[b],
                                 eout_h.at[pl.ds(base, K), pl.ds(col0, H)],
                                 sem_st[b])
            return carry

        lax.fori_loop(0, NCH // 2, step, 0)
        wait_eout(0)
        wait_eout(1)
        plsc.subcore_barrier()

        gxv = tuple(gxs_v[pl.ds(l * L, L)] for l in range(H // L))
        bxvr = tuple(bxv[pl.ds(l * L, L)] for l in range(H // L))
        for j in range(NNCH):
            n0 = n0t + j * RN
            pltpu.sync_copy(accN.at[pl.ds(n0, RN)], anb)
            pltpu.sync_copy(accD.at[pl.ds(n0, RN)], adb)
            pltpu.sync_copy(ax_h.at[pl.ds(n0, RN), pl.ds(col0, H)], axb)
            pltpu.sync_copy(x_h.at[pl.ds(n0, RN), pl.ds(col0, H)], xbuf)

            def nrow(kk, rcarry):
                for l in range(H // L):
                    sl = pl.ds(l * L, L)
                    aggr = anb[kk, sl] / (adb[kk, sl] + 1e-6)
                    u = axb[kk, sl] + aggr
                    xob[kk, sl] = xbuf[kk, sl] + jnp.maximum(
                        u * gxv[l] + bxvr[l], 0.0)
                return rcarry

            lax.fori_loop(0, RN, nrow, 0)
            pltpu.sync_copy(xob, xout_h.at[pl.ds(n0, RN), pl.ds(col0, H)])

    return k(ax, bxt, dxt, ext, ce, e, x, dst, d2, s2, gxs, bxb, ges, beb)


def kernel(x, e, edge_index, WA, bA, WB, bB, WC, bC, WD, bD, WE, bE,
           gamma_x, beta_x, gamma_e, beta_e):
    src = edge_index[0].astype(jnp.int32)
    dst = edge_index[1].astype(jnp.int32)
    d2 = jnp.concatenate([2 * dst, 2 * dst + 1])
    s2 = jnp.concatenate([2 * src, 2 * src + 1])

    ax, bx, dx, ex = _proj_nodes(x, WA, bA, WB, bB, WD, bD, WE, bE)
    ce = _proj_edges(e, WC, bC)

    bxt = bx.reshape(2 * N, H)
    dxt = dx.reshape(2 * N, H)
    ext = ex.reshape(2 * N, H)
    gxs = gamma_x * jnp.float32(BN_INV)
    ges = gamma_e * jnp.float32(BN_INV)

    x_out, e_out = _sc_gnn(ax, bxt, dxt, ext, ce, e, x, dst, d2, s2,
                           gxs, beta_x, ges, beta_e)
    return x_out, e_out
